# baseline (speedup 1.0000x reference)
import functools

import jax
import jax.numpy as jnp
from jax import lax
from jax.experimental import pallas as pl
from jax.experimental.pallas import tpu as pltpu
from jax.experimental.pallas import tpu_sc as plsc

NC = 2
NS = 16
NW = NC * NS
C = 80
NB = 4


def _agg_kernel(npad, d, k0, k1):
    mesh = plsc.VectorSubcoreMesh(core_axis_name="c", subcore_axis_name="s")
    rows_per_tile = npad // NS
    qmax = max(k0, k1) // 8
    assert k0 % 64 == 0 and k1 % 64 == 0

    @functools.partial(
        pl.kernel,
        out_type=jax.ShapeDtypeStruct((NC, npad, d), jnp.float32),
        mesh=mesh,
        scratch_types=[
            pltpu.VMEM((qmax, C), jnp.int32),
            pltpu.VMEM((qmax, C), jnp.int32),
            [pltpu.VMEM((C, d), jnp.float32) for _ in range(NB)],
            pltpu.VMEM_SHARED((npad, d), jnp.float32),
            [pltpu.SemaphoreType.DMA for _ in range(NB)],
            [pltpu.SemaphoreType.DMA for _ in range(NB)],
        ],
    )
    def agg(h_hbm, src_hbm, dst_hbm, zero_hbm, out_hbm,
            src_v, dst_v, bufs, acc, gsems, ssems):
        cid = lax.axis_index("c")
        sid = lax.axis_index("s")
        sl = pl.ds(sid * rows_per_tile, rows_per_tile)
        pltpu.sync_copy(zero_hbm.at[sl], acc.at[sl])
        plsc.subcore_barrier()

        def run_core(kc, base):
            qs = kc // 8

            def run_stage(q, carry):
                row0 = base + q * qs
                pltpu.sync_copy(src_hbm.at[pl.ds(row0, qs)],
                                src_v.at[pl.ds(0, qs)])
                pltpu.sync_copy(dst_hbm.at[pl.ds(row0, qs)],
                                dst_v.at[pl.ds(0, qs)])

                def body(it, carry2):
                    for p in range(NB):
                        j = it * NB + p
                        pn = (p + 1) % NB

                        @pl.when(j >= NB)
                        def _():
                            pltpu.make_async_copy(
                                bufs[p], acc.at[dst_v.at[0]], ssems[p]).wait()

                        @pl.when(j < qs)
                        def _():
                            pltpu.async_copy(h_hbm.at[src_v.at[j]], bufs[p],
                                             gsems[p])

                        @pl.when((j >= NB - 1) & (j < qs + NB - 1))
                        def _():
                            pltpu.make_async_copy(
                                h_hbm.at[src_v.at[0]], bufs[pn],
                                gsems[pn]).wait()
                            pltpu.async_copy(
                                bufs[pn], acc.at[dst_v.at[j - NB + 1]],
                                ssems[pn], add=True)
                    return carry2

                lax.fori_loop(0, (qs + NB) // NB, body, 0)
                return carry

            lax.fori_loop(0, 8, run_stage, 0)

        @pl.when(cid == 0)
        def _():
            run_core(k0, sid * k0)

        @pl.when(cid == 1)
        def _():
            run_core(k1, NS * k0 + sid * k1)

        plsc.subcore_barrier()
        pltpu.sync_copy(acc.at[sl], out_hbm.at[cid, sl])

    return agg


def _deg_kernel(npad, k):
    mesh = plsc.VectorSubcoreMesh(core_axis_name="c", subcore_axis_name="s")
    per_tile = npad // NS

    @functools.partial(
        pl.kernel,
        out_type=jax.ShapeDtypeStruct((NC, npad), jnp.float32),
        mesh=mesh,
        scratch_types=[
            pltpu.VMEM((k, C), jnp.int32),
            pltpu.VMEM((C,), jnp.float32),
            pltpu.VMEM_SHARED((npad,), jnp.float32),
        ],
    )
    def deg(dst_hbm, zero_hbm, out_hbm, dst_v, ones_v, acc):
        cid = lax.axis_index("c")
        sid = lax.axis_index("s")
        wid = sid * NC + cid
        sl = pl.ds(sid * per_tile, per_tile)
        pltpu.sync_copy(zero_hbm.at[sl], acc.at[sl])
        pltpu.sync_copy(dst_hbm.at[pl.ds(wid * k, k)], dst_v)
        for i in range(C // 16):
            ones_v[pl.ds(i * 16, 16)] = jnp.ones((16,), jnp.float32)
        plsc.subcore_barrier()

        def body(j, carry):
            pltpu.sync_copy(ones_v, acc.at[dst_v.at[j]], add=True)
            return carry

        lax.fori_loop(0, k, body, 0)
        plsc.subcore_barrier()
        pltpu.sync_copy(acc.at[sl], out_hbm.at[cid, sl])

    return deg


def _tc_pre(x_p, W1, deg2d, block):
    npad, d = x_p.shape

    def body(x_ref, w_ref, deg_ref, out_ref):
        h = jnp.dot(x_ref[...], w_ref[...], preferred_element_type=jnp.float32)
        out_ref[...] = h * lax.rsqrt(deg_ref[...])

    return pl.pallas_call(
        body,
        grid=(npad // block,),
        in_specs=[
            pl.BlockSpec((block, d), lambda i: (i, 0)),
            pl.BlockSpec((d, d), lambda i: (0, 0)),
            pl.BlockSpec((block, 1), lambda i: (i, 0)),
        ],
        out_specs=pl.BlockSpec((block, d), lambda i: (i, 0)),
        out_shape=jax.ShapeDtypeStruct((npad, d), jnp.float32),
    )(x_p, W1, deg2d)


def _tc_mid(aggp, hp, deg2d, b_2d, a_2d, W2, block):
    _, npad, d = aggp.shape

    def body(agg_ref, hp_ref, deg_ref, b_ref, a_ref, w_ref, out_ref):
        dinv = lax.rsqrt(deg_ref[...])
        s = agg_ref[0] + agg_ref[1] + hp_ref[...]
        z = s * dinv + b_ref[...]
        p = jnp.where(z > 0, z, a_ref[...] * z)
        h2 = jnp.dot(p, w_ref[...], preferred_element_type=jnp.float32)
        out_ref[...] = h2 * dinv

    return pl.pallas_call(
        body,
        grid=(npad // block,),
        in_specs=[
            pl.BlockSpec((2, block, d), lambda i: (0, i, 0)),
            pl.BlockSpec((block, d), lambda i: (i, 0)),
            pl.BlockSpec((block, 1), lambda i: (i, 0)),
            pl.BlockSpec((1, d), lambda i: (0, 0)),
            pl.BlockSpec((1, d), lambda i: (0, 0)),
            pl.BlockSpec((d, d), lambda i: (0, 0)),
        ],
        out_specs=pl.BlockSpec((block, d), lambda i: (i, 0)),
        out_shape=jax.ShapeDtypeStruct((npad, d), jnp.float32),
    )(aggp, hp, deg2d, b_2d, a_2d, W2)


def _tc_post(aggp, hp, deg2d, b_2d, a_2d, block):
    _, npad, d = aggp.shape

    def body(agg_ref, hp_ref, deg_ref, b_ref, a_ref, out_ref):
        dinv = lax.rsqrt(deg_ref[...])
        z = (agg_ref[0] + agg_ref[1] + hp_ref[...]) * dinv + b_ref[...]
        out_ref[...] = jnp.where(z > 0, z, a_ref[...] * z)

    return pl.pallas_call(
        body,
        grid=(npad // block,),
        in_specs=[
            pl.BlockSpec((2, block, d), lambda i: (0, i, 0)),
            pl.BlockSpec((block, d), lambda i: (i, 0)),
            pl.BlockSpec((block, 1), lambda i: (i, 0)),
            pl.BlockSpec((1, d), lambda i: (0, 0)),
            pl.BlockSpec((1, d), lambda i: (0, 0)),
        ],
        out_specs=pl.BlockSpec((block, d), lambda i: (i, 0)),
        out_shape=jax.ShapeDtypeStruct((npad, d), jnp.float32),
    )(aggp, hp, deg2d, b_2d, a_2d)


def kernel(x, edge_index, W1, b1, a1, W2, b2, a2):
    n, d = x.shape
    e = edge_index.shape[1]
    npad = 10240 if n == 10000 else ((n + 8 * NW) // (8 * NW)) * (8 * NW)
    k = ((e + C * NW - 1) // (C * NW) + 15) // 16 * 16
    epad = k * C * NW
    k1 = max(64, (2 * k) // 4 // 64 * 64)
    k0 = 2 * k - k1
    block = 512

    src = edge_index[0].astype(jnp.int32)
    dst = edge_index[1].astype(jnp.int32)
    pad = jnp.full((epad - e,), n, dtype=jnp.int32)
    src_p = jnp.concatenate([src, pad]).reshape(epad // C, C)
    dst_p = jnp.concatenate([dst, pad]).reshape(epad // C, C)
    x_p = jnp.zeros((npad, d), jnp.float32).at[:n].set(x)
    z1 = jnp.zeros((npad,), jnp.float32)
    z2 = jnp.zeros((npad, d), jnp.float32)

    degp = _deg_kernel(npad, k)(dst_p, z1)
    deg2d = (degp[0] + degp[1] + 1.0).reshape(npad, 1)

    agg = _agg_kernel(npad, d, k0, k1)
    h1p = _tc_pre(x_p, W1, deg2d, block)
    a1g = agg(h1p, src_p, dst_p, z2)
    h2p = _tc_mid(a1g, h1p, deg2d, b1.reshape(1, d), a1.reshape(1, d),
                  W2, block)
    a2g = agg(h2p, src_p, dst_p, z2)
    out = _tc_post(a2g, h2p, deg2d, b2.reshape(1, d), a2.reshape(1, d), block)
    return out[:n]
